# baseline (speedup 1.0000x reference)
import jax
import jax.numpy as jnp
from jax import lax
from jax.experimental import pallas as pl
from jax.experimental.pallas import tpu as pltpu
from jax.experimental.pallas import tpu_sc as plsc

N_REFLN = 2_000_000
D = 16
GRID_DIM = 161
NC = 2
NS = 16
NW = NC * NS
B = 128
K = 4
SB = K * B
NBATCH = N_REFLN // B
PER_W = NBATCH // NW
REM = NBATCH - PER_W * NW
NSB = PER_W // K
PAIRS = NSB // 2


import numpy as np

_DNUMS = lax.GatherDimensionNumbers(
    offset_dims=(), collapsed_slice_dims=(0,), start_index_map=(0,)
)


def _dg(v, idx):
    return lax.gather(
        v,
        idx[:, None],
        dimension_numbers=_DNUMS,
        slice_sizes=(1,),
        mode=lax.GatherScatterMode.PROMISE_IN_BOUNDS,
    )


def _lane_consts():
    iota = lax.iota(jnp.int32, 16)
    out = []
    for comp in range(3):
        p = iota * 3 + comp
        out.append((p % 16, p < 16, p < 32))
    return out


def _extract(v0, v1, v2, lc):
    g, in0, in1 = lc
    return jnp.where(
        in0, _dg(v0, g), jnp.where(in1, _dg(v1, g), _dg(v2, g))
    )


def _lin16(href, flat_base, lane3):
    h0 = plsc.load_gather(href, [lane3 + flat_base])
    h1 = plsc.load_gather(href, [lane3 + (flat_base + 1)])
    h2 = plsc.load_gather(href, [lane3 + (flat_base + 2)])
    return h0 * (GRID_DIM * GRID_DIM) + h1 * GRID_DIM + h2


KB8 = K * B * 8
HALF = NBATCH * B * 8


def _sc_body(h_hbm, grid_hbm, src_hbm, out_hbm,
             h_v, lin_v, gid_v, rows_v, rt_v,
             sem_g0, sem_g1, sem_s0, sem_s1, sem_o0, sem_o1):
    c = lax.axis_index("c")
    s = lax.axis_index("s")
    wid = s * NC + c
    b0 = wid * PER_W
    lane = lax.iota(jnp.int32, 16)
    cvec = (lane // 8) * KB8 + (lane % 8) * B
    ccol = [lane * 0 + comp for comp in range(16)]

    lv = (lin_v.at[0], lin_v.at[1])
    gv = (gid_v.at[0], gid_v.at[1])
    rv = (rows_v.at[0], rows_v.at[1])
    rt = (rt_v.at[0], rt_v.at[1])
    sg = (sem_g0, sem_g1)
    ss = (sem_s0, sem_s1)
    so = (sem_o0, sem_o1)

    def load_h(sb, n_sb):
        pltpu.sync_copy(
            h_hbm.at[pl.ds(sb * (3 * SB), n_sb * 3 * SB)],
            h_v.at[pl.ds(0, n_sb * 3 * SB)],
        )

    def lin(p):
        hb = p * 3 * SB
        for k in range(K):
            for j in range(B // 16):
                h0 = h_v[pl.ds(hb + k * 3 * B + j * 16, 16)]
                h1 = h_v[pl.ds(hb + k * 3 * B + B + j * 16, 16)]
                h2 = h_v[pl.ds(hb + k * 3 * B + 2 * B + j * 16, 16)]
                lv[p][pl.ds(k * B + j * 16, 16)] = (
                    h0 * (GRID_DIM * GRID_DIM) + h1 * GRID_DIM + h2
                )

    def fire_grid(p):
        return [
            pltpu.async_copy(
                grid_hbm.at[lv[p].at[pl.ds(k * B, B)]],
                gv[p].at[pl.ds(k * B, B)],
                sg[p],
            )
            for k in range(K)
        ]

    def fire_src(p):
        return [
            pltpu.async_copy(
                src_hbm.at[gv[p].at[pl.ds(k * B, B)]],
                rv[p].at[pl.ds(k * B, B)],
                ss[p],
            )
            for k in range(K)
        ]

    def transpose(p):
        for q in range(SB // 16):
            k, lbase = q // (B // 16), (q % (B // 16)) * 16
            ridx = lane + q * 16
            for comp in range(16):
                vals = plsc.load_gather(rv[p], [ridx, ccol[comp]])
                dst = (comp // 8) * KB8 + k * (8 * B) + (comp % 8) * B + lbase
                rt[p][pl.ds(dst, 16)] = vals

    def fire_out(sb, p):
        base = sb * K * (8 * B)
        return [
            pltpu.async_copy(
                rt[p].at[pl.ds(0, KB8)],
                out_hbm.at[pl.ds(base, KB8)],
                so[p],
            ),
            pltpu.async_copy(
                rt[p].at[pl.ds(KB8, KB8)],
                out_hbm.at[pl.ds(HALF + base, KB8)],
                so[p],
            ),
        ]

    def drain(copies):
        for cp in copies:
            cp.wait()

    def pair_body(t, carry):
        sb_a = b0 // K + 2 * t
        sb_b = sb_a + 1
        load_h(sb_a, 2)
        lin(0)
        ga = fire_grid(0)
        lin(1)
        drain(ga)
        sa = fire_src(0)
        gb = fire_grid(1)
        drain(sa)
        transpose(0)
        oa = fire_out(sb_a, 0)
        drain(gb)
        sb_ = fire_src(1)
        drain(sb_)
        transpose(1)
        ob = fire_out(sb_b, 1)
        drain(oa)
        drain(ob)
        return carry

    lax.fori_loop(0, PAIRS, pair_body, 0)

    if NSB % 2:
        sb_last = b0 // K + 2 * PAIRS
        load_h(sb_last, 1)
        lin(0)
        drain(fire_grid(0))
        drain(fire_src(0))
        transpose(0)
        drain(fire_out(sb_last, 0))

    @pl.when(wid < REM)
    def _():
        m = NBATCH - REM + wid
        pltpu.sync_copy(
            h_hbm.at[pl.ds(m * (3 * B), 3 * B)], h_v.at[pl.ds(0, 3 * B)]
        )
        for j in range(B // 16):
            h0 = h_v[pl.ds(j * 16, 16)]
            h1 = h_v[pl.ds(B + j * 16, 16)]
            h2 = h_v[pl.ds(2 * B + j * 16, 16)]
            lv[0][pl.ds(j * 16, 16)] = (
                h0 * (GRID_DIM * GRID_DIM) + h1 * GRID_DIM + h2
            )
        pltpu.async_copy(
            grid_hbm.at[lv[0].at[pl.ds(0, B)]], gv[0].at[pl.ds(0, B)], sg[0]
        ).wait()
        pltpu.async_copy(
            src_hbm.at[gv[0].at[pl.ds(0, B)]], rv[0].at[pl.ds(0, B)], ss[0]
        ).wait()
        for l in range(B):
            v = rv[0].at[l][...]
            plsc.store_scatter(rt[0], [cvec + l], v)
        base = m * (8 * B)
        pltpu.sync_copy(
            rt[0].at[pl.ds(0, 8 * B)], out_hbm.at[pl.ds(base, 8 * B)]
        )
        pltpu.sync_copy(
            rt[0].at[pl.ds(KB8, 8 * B)],
            out_hbm.at[pl.ds(HALF + base, 8 * B)],
        )


def kernel(source, H, reflection_id_grid):
    h_blk = (
        H.astype(jnp.int32)
        .reshape(NBATCH, B, 3)
        .transpose(0, 2, 1)
        .reshape(-1)
    )
    grid_flat = reflection_id_grid.reshape(-1)
    mesh = plsc.VectorSubcoreMesh(core_axis_name="c", subcore_axis_name="s")
    run = pl.kernel(
        _sc_body,
        mesh=mesh,
        compiler_params=pltpu.CompilerParams(
            use_tc_tiling_on_sc=False, needs_layout_passes=False
        ),
        out_type=jax.ShapeDtypeStruct((N_REFLN * D,), jnp.float32),
        scratch_types=[
            pltpu.VMEM((2 * 3 * SB,), jnp.int32),
            pltpu.VMEM((2, SB), jnp.int32),
            pltpu.VMEM((2, SB), jnp.int32),
            pltpu.VMEM((2, SB, D), jnp.float32),
            pltpu.VMEM((2, 2 * KB8), jnp.float32),
            pltpu.SemaphoreType.DMA,
            pltpu.SemaphoreType.DMA,
            pltpu.SemaphoreType.DMA,
            pltpu.SemaphoreType.DMA,
            pltpu.SemaphoreType.DMA,
            pltpu.SemaphoreType.DMA,
        ],
    )
    out1d = run(h_blk, grid_flat, source)
    return (
        out1d.reshape(2, NBATCH, 8, B)
        .transpose(1, 3, 0, 2)
        .reshape(N_REFLN, D)
    )


# speedup vs baseline: 1.0007x; 1.0007x over previous
import jax
import jax.numpy as jnp
from jax import lax
from jax.experimental import pallas as pl
from jax.experimental.pallas import tpu as pltpu
from jax.experimental.pallas import tpu_sc as plsc

N_REFLN = 2_000_000
D = 16
GRID_DIM = 161
NC = 2
NS = 16
NW = NC * NS
B = 128
K = 4
SB = K * B
NBATCH = N_REFLN // B
PER_W = NBATCH // NW
REM = NBATCH - PER_W * NW
NSB = PER_W // K
PAIRS = NSB // 2


import numpy as np

_DNUMS = lax.GatherDimensionNumbers(
    offset_dims=(), collapsed_slice_dims=(0,), start_index_map=(0,)
)


def _dg(v, idx):
    return lax.gather(
        v,
        idx[:, None],
        dimension_numbers=_DNUMS,
        slice_sizes=(1,),
        mode=lax.GatherScatterMode.PROMISE_IN_BOUNDS,
    )


def _lane_consts():
    iota = lax.iota(jnp.int32, 16)
    out = []
    for comp in range(3):
        p = iota * 3 + comp
        out.append((p % 16, p < 16, p < 32))
    return out


def _extract(v0, v1, v2, lc):
    g, in0, in1 = lc
    return jnp.where(
        in0, _dg(v0, g), jnp.where(in1, _dg(v1, g), _dg(v2, g))
    )


def _lin16(href, flat_base, lane3):
    h0 = plsc.load_gather(href, [lane3 + flat_base])
    h1 = plsc.load_gather(href, [lane3 + (flat_base + 1)])
    h2 = plsc.load_gather(href, [lane3 + (flat_base + 2)])
    return h0 * (GRID_DIM * GRID_DIM) + h1 * GRID_DIM + h2


KB8 = K * B * 8
HALF = NBATCH * B * 8


def _sc_body(h_hbm, grid_hbm, src_hbm, out_hbm,
             h_v, lin_v, gid_v, rows_v, rt_v,
             sem_g0, sem_g1, sem_s0, sem_s1, sem_o0, sem_o1):
    c = lax.axis_index("c")
    s = lax.axis_index("s")
    wid = s * NC + c
    b0 = wid * PER_W
    lane = lax.iota(jnp.int32, 16)
    cvec = (lane // 8) * KB8 + (lane % 8) * B
    ccol = [lane * 0 + comp for comp in range(16)]

    lv = (lin_v.at[0], lin_v.at[1])
    gv = (gid_v.at[0], gid_v.at[1])
    rv = (rows_v.at[0], rows_v.at[1])
    rt = (rt_v.at[0], rt_v.at[1])
    sg = (sem_g0, sem_g1)
    ss = (sem_s0, sem_s1)
    so = (sem_o0, sem_o1)

    def load_h(sb, n_sb):
        pltpu.sync_copy(
            h_hbm.at[pl.ds(sb * (3 * SB), n_sb * 3 * SB)],
            h_v.at[pl.ds(0, n_sb * 3 * SB)],
        )

    def lin(p):
        hb = p * 3 * SB
        for k in range(K):
            for j in range(B // 16):
                h0 = h_v[pl.ds(hb + k * 3 * B + j * 16, 16)]
                h1 = h_v[pl.ds(hb + k * 3 * B + B + j * 16, 16)]
                h2 = h_v[pl.ds(hb + k * 3 * B + 2 * B + j * 16, 16)]
                lv[p][pl.ds(k * B + j * 16, 16)] = (
                    h0 * (GRID_DIM * GRID_DIM) + h1 * GRID_DIM + h2
                )

    def fire_grid(p):
        return [pltpu.async_copy(grid_hbm.at[lv[p]], gv[p], sg[p])]

    def fire_src(p):
        return [pltpu.async_copy(src_hbm.at[gv[p]], rv[p], ss[p])]

    def transpose(p):
        for q in range(SB // 16):
            k, lbase = q // (B // 16), (q % (B // 16)) * 16
            ridx = lane + q * 16
            for comp in range(16):
                vals = plsc.load_gather(rv[p], [ridx, ccol[comp]])
                dst = (comp // 8) * KB8 + k * (8 * B) + (comp % 8) * B + lbase
                rt[p][pl.ds(dst, 16)] = vals

    def fire_out(sb, p):
        base = sb * K * (8 * B)
        return [
            pltpu.async_copy(
                rt[p].at[pl.ds(0, KB8)],
                out_hbm.at[pl.ds(base, KB8)],
                so[p],
            ),
            pltpu.async_copy(
                rt[p].at[pl.ds(KB8, KB8)],
                out_hbm.at[pl.ds(HALF + base, KB8)],
                so[p],
            ),
        ]

    def drain(copies):
        for cp in copies:
            cp.wait()

    def pair_body(t, carry):
        sb_a = b0 // K + 2 * t
        sb_b = sb_a + 1
        load_h(sb_a, 2)
        lin(0)
        ga = fire_grid(0)
        lin(1)
        drain(ga)
        sa = fire_src(0)
        gb = fire_grid(1)
        drain(sa)
        transpose(0)
        oa = fire_out(sb_a, 0)
        drain(gb)
        sb_ = fire_src(1)
        drain(sb_)
        transpose(1)
        ob = fire_out(sb_b, 1)
        drain(oa)
        drain(ob)
        return carry

    lax.fori_loop(0, PAIRS, pair_body, 0)

    if NSB % 2:
        sb_last = b0 // K + 2 * PAIRS
        load_h(sb_last, 1)
        lin(0)
        drain(fire_grid(0))
        drain(fire_src(0))
        transpose(0)
        drain(fire_out(sb_last, 0))

    @pl.when(wid < REM)
    def _():
        m = NBATCH - REM + wid
        pltpu.sync_copy(
            h_hbm.at[pl.ds(m * (3 * B), 3 * B)], h_v.at[pl.ds(0, 3 * B)]
        )
        for j in range(B // 16):
            h0 = h_v[pl.ds(j * 16, 16)]
            h1 = h_v[pl.ds(B + j * 16, 16)]
            h2 = h_v[pl.ds(2 * B + j * 16, 16)]
            lv[0][pl.ds(j * 16, 16)] = (
                h0 * (GRID_DIM * GRID_DIM) + h1 * GRID_DIM + h2
            )
        pltpu.async_copy(
            grid_hbm.at[lv[0].at[pl.ds(0, B)]], gv[0].at[pl.ds(0, B)], sg[0]
        ).wait()
        pltpu.async_copy(
            src_hbm.at[gv[0].at[pl.ds(0, B)]], rv[0].at[pl.ds(0, B)], ss[0]
        ).wait()
        for l in range(B):
            v = rv[0].at[l][...]
            plsc.store_scatter(rt[0], [cvec + l], v)
        base = m * (8 * B)
        pltpu.sync_copy(
            rt[0].at[pl.ds(0, 8 * B)], out_hbm.at[pl.ds(base, 8 * B)]
        )
        pltpu.sync_copy(
            rt[0].at[pl.ds(KB8, 8 * B)],
            out_hbm.at[pl.ds(HALF + base, 8 * B)],
        )


def kernel(source, H, reflection_id_grid):
    h_blk = (
        H.astype(jnp.int32)
        .reshape(NBATCH, B, 3)
        .transpose(0, 2, 1)
        .reshape(-1)
    )
    grid_flat = reflection_id_grid.reshape(-1)
    mesh = plsc.VectorSubcoreMesh(core_axis_name="c", subcore_axis_name="s")
    run = pl.kernel(
        _sc_body,
        mesh=mesh,
        compiler_params=pltpu.CompilerParams(
            use_tc_tiling_on_sc=False, needs_layout_passes=False
        ),
        out_type=jax.ShapeDtypeStruct((N_REFLN * D,), jnp.float32),
        scratch_types=[
            pltpu.VMEM((2 * 3 * SB,), jnp.int32),
            pltpu.VMEM((2, SB), jnp.int32),
            pltpu.VMEM((2, SB), jnp.int32),
            pltpu.VMEM((2, SB, D), jnp.float32),
            pltpu.VMEM((2, 2 * KB8), jnp.float32),
            pltpu.SemaphoreType.DMA,
            pltpu.SemaphoreType.DMA,
            pltpu.SemaphoreType.DMA,
            pltpu.SemaphoreType.DMA,
            pltpu.SemaphoreType.DMA,
            pltpu.SemaphoreType.DMA,
        ],
    )
    out1d = run(h_blk, grid_flat, source)
    return (
        out1d.reshape(2, NBATCH, 8, B)
        .transpose(1, 3, 0, 2)
        .reshape(N_REFLN, D)
    )


# speedup vs baseline: 1.3404x; 1.3395x over previous
import jax
import jax.numpy as jnp
from jax import lax
from jax.experimental import pallas as pl
from jax.experimental.pallas import tpu as pltpu
from jax.experimental.pallas import tpu_sc as plsc

N_REFLN = 2_000_000
D = 16
GRID_DIM = 161
NC = 2
NS = 16
NW = NC * NS
B = 128
K = 4
SB = K * B
NBATCH = N_REFLN // B
PER_W = NBATCH // NW
REM = NBATCH - PER_W * NW
NSB = PER_W // K
NBUF = 4
BODIES = NSB // NBUF
LEFT = NSB - NBUF * BODIES


import numpy as np

_DNUMS = lax.GatherDimensionNumbers(
    offset_dims=(), collapsed_slice_dims=(0,), start_index_map=(0,)
)


def _dg(v, idx):
    return lax.gather(
        v,
        idx[:, None],
        dimension_numbers=_DNUMS,
        slice_sizes=(1,),
        mode=lax.GatherScatterMode.PROMISE_IN_BOUNDS,
    )


def _lane_consts():
    iota = lax.iota(jnp.int32, 16)
    out = []
    for comp in range(3):
        p = iota * 3 + comp
        out.append((p % 16, p < 16, p < 32))
    return out


def _extract(v0, v1, v2, lc):
    g, in0, in1 = lc
    return jnp.where(
        in0, _dg(v0, g), jnp.where(in1, _dg(v1, g), _dg(v2, g))
    )


def _lin16(href, flat_base, lane3):
    h0 = plsc.load_gather(href, [lane3 + flat_base])
    h1 = plsc.load_gather(href, [lane3 + (flat_base + 1)])
    h2 = plsc.load_gather(href, [lane3 + (flat_base + 2)])
    return h0 * (GRID_DIM * GRID_DIM) + h1 * GRID_DIM + h2


KB8 = K * B * 8
HALF = NBATCH * B * 8


def _sc_body(h_hbm, grid_hbm, src_hbm, out_hbm,
             h_v, lin_v, gid_v, rows_v, rt_v,
             sem_g0, sem_g1, sem_g2, sem_g3, sem_s0, sem_s1, sem_s2, sem_s3,
             sem_o0, sem_o1, sem_o2, sem_o3):
    c = lax.axis_index("c")
    s = lax.axis_index("s")
    wid = s * NC + c
    b0 = wid * PER_W
    lane = lax.iota(jnp.int32, 16)
    cvec = (lane // 8) * KB8 + (lane % 8) * B
    ccol = [lane * 0 + comp for comp in range(16)]

    lv = tuple(lin_v.at[i] for i in range(NBUF))
    gv = tuple(gid_v.at[i] for i in range(NBUF))
    rv = tuple(rows_v.at[i] for i in range(NBUF))
    rt = tuple(rt_v.at[i] for i in range(NBUF))
    sg = (sem_g0, sem_g1, sem_g2, sem_g3)
    ss = (sem_s0, sem_s1, sem_s2, sem_s3)
    so = (sem_o0, sem_o1, sem_o2, sem_o3)

    def load_h(sb, n_sb):
        pltpu.sync_copy(
            h_hbm.at[pl.ds(sb * (3 * SB), n_sb * 3 * SB)],
            h_v.at[pl.ds(0, n_sb * 3 * SB)],
        )

    def lin(p):
        hb = p * 3 * SB
        for k in range(K):
            for j in range(B // 16):
                h0 = h_v[pl.ds(hb + k * 3 * B + j * 16, 16)]
                h1 = h_v[pl.ds(hb + k * 3 * B + B + j * 16, 16)]
                h2 = h_v[pl.ds(hb + k * 3 * B + 2 * B + j * 16, 16)]
                lv[p][pl.ds(k * B + j * 16, 16)] = (
                    h0 * (GRID_DIM * GRID_DIM) + h1 * GRID_DIM + h2
                )

    def fire_grid(p):
        return [pltpu.async_copy(grid_hbm.at[lv[p]], gv[p], sg[p])]

    def fire_src(p):
        return [pltpu.async_copy(src_hbm.at[gv[p]], rv[p], ss[p])]

    def transpose(p):
        def grp(q, carry):
            k = q // (B // 16)
            lbase = q * 16 - k * B
            ridx = lane + q * 16
            for comp in range(16):
                vals = plsc.load_gather(rv[p], [ridx, ccol[comp]])
                dst = (comp // 8) * KB8 + k * (8 * B) + (comp % 8) * B + lbase
                rt[p][pl.ds(dst, 16)] = vals
            return carry

        lax.fori_loop(0, SB // 16, grp, 0)

    def fire_out(sb, p):
        base = sb * K * (8 * B)
        return [
            pltpu.async_copy(
                rt[p].at[pl.ds(0, KB8)],
                out_hbm.at[pl.ds(base, KB8)],
                so[p],
            ),
            pltpu.async_copy(
                rt[p].at[pl.ds(KB8, KB8)],
                out_hbm.at[pl.ds(HALF + base, KB8)],
                so[p],
            ),
        ]

    def drain(copies):
        for cp in copies:
            cp.wait()

    def body(t, carry):
        sb0 = b0 // K + NBUF * t
        load_h(sb0, NBUF)
        g, s, o = [], [], []
        for i in range(NBUF):
            lin(i)
            g.append(fire_grid(i))
        for i in range(NBUF):
            drain(g[i])
            s.append(fire_src(i))
        for i in range(NBUF):
            drain(s[i])
            transpose(i)
            o.append(fire_out(sb0 + i, i))
        for i in range(NBUF):
            drain(o[i])
        return carry

    lax.fori_loop(0, BODIES, body, 0)

    for j in range(LEFT):
        sb_last = b0 // K + NBUF * BODIES + j
        load_h(sb_last, 1)
        lin(0)
        drain(fire_grid(0))
        drain(fire_src(0))
        transpose(0)
        drain(fire_out(sb_last, 0))

    @pl.when(wid < REM)
    def _():
        m = NBATCH - REM + wid
        pltpu.sync_copy(
            h_hbm.at[pl.ds(m * (3 * B), 3 * B)], h_v.at[pl.ds(0, 3 * B)]
        )
        for j in range(B // 16):
            h0 = h_v[pl.ds(j * 16, 16)]
            h1 = h_v[pl.ds(B + j * 16, 16)]
            h2 = h_v[pl.ds(2 * B + j * 16, 16)]
            lv[0][pl.ds(j * 16, 16)] = (
                h0 * (GRID_DIM * GRID_DIM) + h1 * GRID_DIM + h2
            )
        pltpu.async_copy(
            grid_hbm.at[lv[0].at[pl.ds(0, B)]], gv[0].at[pl.ds(0, B)], sg[0]
        ).wait()
        pltpu.async_copy(
            src_hbm.at[gv[0].at[pl.ds(0, B)]], rv[0].at[pl.ds(0, B)], ss[0]
        ).wait()
        for l in range(B):
            v = rv[0].at[l][...]
            plsc.store_scatter(rt[0], [cvec + l], v)
        base = m * (8 * B)
        pltpu.sync_copy(
            rt[0].at[pl.ds(0, 8 * B)], out_hbm.at[pl.ds(base, 8 * B)]
        )
        pltpu.sync_copy(
            rt[0].at[pl.ds(KB8, 8 * B)],
            out_hbm.at[pl.ds(HALF + base, 8 * B)],
        )


def kernel(source, H, reflection_id_grid):
    h_blk = (
        H.astype(jnp.int32)
        .reshape(NBATCH, B, 3)
        .transpose(0, 2, 1)
        .reshape(-1)
    )
    grid_flat = reflection_id_grid.reshape(-1)
    mesh = plsc.VectorSubcoreMesh(core_axis_name="c", subcore_axis_name="s")
    run = pl.kernel(
        _sc_body,
        mesh=mesh,
        compiler_params=pltpu.CompilerParams(
            use_tc_tiling_on_sc=False, needs_layout_passes=False
        ),
        out_type=jax.ShapeDtypeStruct((N_REFLN * D,), jnp.float32),
        scratch_types=[
            pltpu.VMEM((NBUF * 3 * SB,), jnp.int32),
            pltpu.VMEM((NBUF, SB), jnp.int32),
            pltpu.VMEM((NBUF, SB), jnp.int32),
            pltpu.VMEM((NBUF, SB, D), jnp.float32),
            pltpu.VMEM((NBUF, 2 * KB8), jnp.float32),
            pltpu.SemaphoreType.DMA,
            pltpu.SemaphoreType.DMA,
            pltpu.SemaphoreType.DMA,
            pltpu.SemaphoreType.DMA,
            pltpu.SemaphoreType.DMA,
            pltpu.SemaphoreType.DMA,
            pltpu.SemaphoreType.DMA,
            pltpu.SemaphoreType.DMA,
            pltpu.SemaphoreType.DMA,
            pltpu.SemaphoreType.DMA,
            pltpu.SemaphoreType.DMA,
            pltpu.SemaphoreType.DMA,
        ],
    )
    out1d = run(h_blk, grid_flat, source)
    return (
        out1d.reshape(2, NBATCH, 8, B)
        .transpose(1, 3, 0, 2)
        .reshape(N_REFLN, D)
    )


# speedup vs baseline: 1.4041x; 1.0475x over previous
import jax
import jax.numpy as jnp
from jax import lax
from jax.experimental import pallas as pl
from jax.experimental.pallas import tpu as pltpu
from jax.experimental.pallas import tpu_sc as plsc

N_REFLN = 2_000_000
D = 16
GRID_DIM = 161
NC = 2
NS = 16
NW = NC * NS
B = 128
K = 4
SB = K * B
NBATCH = N_REFLN // B
PER_W = NBATCH // NW
REM = NBATCH - PER_W * NW
NSB = PER_W // K
NBUF = 6
BODIES = NSB // NBUF
LEFT = NSB - NBUF * BODIES


import numpy as np

_DNUMS = lax.GatherDimensionNumbers(
    offset_dims=(), collapsed_slice_dims=(0,), start_index_map=(0,)
)


def _dg(v, idx):
    return lax.gather(
        v,
        idx[:, None],
        dimension_numbers=_DNUMS,
        slice_sizes=(1,),
        mode=lax.GatherScatterMode.PROMISE_IN_BOUNDS,
    )


def _lane_consts():
    iota = lax.iota(jnp.int32, 16)
    out = []
    for comp in range(3):
        p = iota * 3 + comp
        out.append((p % 16, p < 16, p < 32))
    return out


def _extract(v0, v1, v2, lc):
    g, in0, in1 = lc
    return jnp.where(
        in0, _dg(v0, g), jnp.where(in1, _dg(v1, g), _dg(v2, g))
    )


def _lin16(href, flat_base, lane3):
    h0 = plsc.load_gather(href, [lane3 + flat_base])
    h1 = plsc.load_gather(href, [lane3 + (flat_base + 1)])
    h2 = plsc.load_gather(href, [lane3 + (flat_base + 2)])
    return h0 * (GRID_DIM * GRID_DIM) + h1 * GRID_DIM + h2


KB8 = K * B * 8
HALF = NBATCH * B * 8


def _sc_body(h_hbm, grid_hbm, src_hbm, out_hbm,
             h_v, lin_v, gid_v, rows_v, rt_v,
             sem_g0, sem_g1, sem_g2, sem_g3, sem_g4, sem_g5,
             sem_s0, sem_s1, sem_s2, sem_s3, sem_s4, sem_s5,
             sem_o0, sem_o1, sem_o2, sem_o3, sem_o4, sem_o5):
    c = lax.axis_index("c")
    s = lax.axis_index("s")
    wid = s * NC + c
    b0 = wid * PER_W
    lane = lax.iota(jnp.int32, 16)
    cvec = (lane // 8) * KB8 + (lane % 8) * B
    ccol = [lane * 0 + comp for comp in range(16)]

    lv = tuple(lin_v.at[i] for i in range(NBUF))
    gv = tuple(gid_v.at[i] for i in range(NBUF))
    rv = tuple(rows_v.at[i] for i in range(NBUF))
    rt = tuple(rt_v.at[i] for i in range(NBUF))
    sg = (sem_g0, sem_g1, sem_g2, sem_g3, sem_g4, sem_g5)
    ss = (sem_s0, sem_s1, sem_s2, sem_s3, sem_s4, sem_s5)
    so = (sem_o0, sem_o1, sem_o2, sem_o3, sem_o4, sem_o5)

    def load_h(sb, n_sb):
        pltpu.sync_copy(
            h_hbm.at[pl.ds(sb * (3 * SB), n_sb * 3 * SB)],
            h_v.at[pl.ds(0, n_sb * 3 * SB)],
        )

    def lin(p):
        hb = p * 3 * SB
        for k in range(K):
            for j in range(B // 16):
                h0 = h_v[pl.ds(hb + k * 3 * B + j * 16, 16)]
                h1 = h_v[pl.ds(hb + k * 3 * B + B + j * 16, 16)]
                h2 = h_v[pl.ds(hb + k * 3 * B + 2 * B + j * 16, 16)]
                lv[p][pl.ds(k * B + j * 16, 16)] = (
                    h0 * (GRID_DIM * GRID_DIM) + h1 * GRID_DIM + h2
                )

    def fire_grid(p):
        return [pltpu.async_copy(grid_hbm.at[lv[p]], gv[p], sg[p])]

    def fire_src(p):
        return [pltpu.async_copy(src_hbm.at[gv[p]], rv[p], ss[p])]

    def transpose(p):
        def grp(q, carry):
            k = q // (B // 16)
            lbase = q * 16 - k * B
            ridx = lane + q * 16
            for comp in range(16):
                vals = plsc.load_gather(rv[p], [ridx, ccol[comp]])
                dst = (comp // 8) * KB8 + k * (8 * B) + (comp % 8) * B + lbase
                rt[p][pl.ds(dst, 16)] = vals
            return carry

        lax.fori_loop(0, SB // 16, grp, 0)

    def fire_out(sb, p):
        base = sb * K * (8 * B)
        return [
            pltpu.async_copy(
                rt[p].at[pl.ds(0, KB8)],
                out_hbm.at[pl.ds(base, KB8)],
                so[p],
            ),
            pltpu.async_copy(
                rt[p].at[pl.ds(KB8, KB8)],
                out_hbm.at[pl.ds(HALF + base, KB8)],
                so[p],
            ),
        ]

    def drain(copies):
        for cp in copies:
            cp.wait()

    def body(t, carry):
        sb0 = b0 // K + NBUF * t
        load_h(sb0, NBUF)
        g, s, o = [], [], []
        for i in range(NBUF):
            lin(i)
            g.append(fire_grid(i))
        for i in range(NBUF):
            drain(g[i])
            s.append(fire_src(i))
        for i in range(NBUF):
            drain(s[i])
            transpose(i)
            o.append(fire_out(sb0 + i, i))
        for i in range(NBUF):
            drain(o[i])
        return carry

    lax.fori_loop(0, BODIES, body, 0)

    for j in range(LEFT):
        sb_last = b0 // K + NBUF * BODIES + j
        load_h(sb_last, 1)
        lin(0)
        drain(fire_grid(0))
        drain(fire_src(0))
        transpose(0)
        drain(fire_out(sb_last, 0))

    @pl.when(wid < REM)
    def _():
        m = NBATCH - REM + wid
        pltpu.sync_copy(
            h_hbm.at[pl.ds(m * (3 * B), 3 * B)], h_v.at[pl.ds(0, 3 * B)]
        )
        for j in range(B // 16):
            h0 = h_v[pl.ds(j * 16, 16)]
            h1 = h_v[pl.ds(B + j * 16, 16)]
            h2 = h_v[pl.ds(2 * B + j * 16, 16)]
            lv[0][pl.ds(j * 16, 16)] = (
                h0 * (GRID_DIM * GRID_DIM) + h1 * GRID_DIM + h2
            )
        pltpu.async_copy(
            grid_hbm.at[lv[0].at[pl.ds(0, B)]], gv[0].at[pl.ds(0, B)], sg[0]
        ).wait()
        pltpu.async_copy(
            src_hbm.at[gv[0].at[pl.ds(0, B)]], rv[0].at[pl.ds(0, B)], ss[0]
        ).wait()
        for l in range(B):
            v = rv[0].at[l][...]
            plsc.store_scatter(rt[0], [cvec + l], v)
        base = m * (8 * B)
        pltpu.sync_copy(
            rt[0].at[pl.ds(0, 8 * B)], out_hbm.at[pl.ds(base, 8 * B)]
        )
        pltpu.sync_copy(
            rt[0].at[pl.ds(KB8, 8 * B)],
            out_hbm.at[pl.ds(HALF + base, 8 * B)],
        )


def kernel(source, H, reflection_id_grid):
    h_blk = (
        H.astype(jnp.int32)
        .reshape(NBATCH, B, 3)
        .transpose(0, 2, 1)
        .reshape(-1)
    )
    grid_flat = reflection_id_grid.reshape(-1)
    mesh = plsc.VectorSubcoreMesh(core_axis_name="c", subcore_axis_name="s")
    run = pl.kernel(
        _sc_body,
        mesh=mesh,
        compiler_params=pltpu.CompilerParams(
            use_tc_tiling_on_sc=False, needs_layout_passes=False
        ),
        out_type=jax.ShapeDtypeStruct((N_REFLN * D,), jnp.float32),
        scratch_types=[
            pltpu.VMEM((NBUF * 3 * SB,), jnp.int32),
            pltpu.VMEM((NBUF, SB), jnp.int32),
            pltpu.VMEM((NBUF, SB), jnp.int32),
            pltpu.VMEM((NBUF, SB, D), jnp.float32),
            pltpu.VMEM((NBUF, 2 * KB8), jnp.float32),
            pltpu.SemaphoreType.DMA,
            pltpu.SemaphoreType.DMA,
            pltpu.SemaphoreType.DMA,
            pltpu.SemaphoreType.DMA,
            pltpu.SemaphoreType.DMA,
            pltpu.SemaphoreType.DMA,
            pltpu.SemaphoreType.DMA,
            pltpu.SemaphoreType.DMA,
            pltpu.SemaphoreType.DMA,
            pltpu.SemaphoreType.DMA,
            pltpu.SemaphoreType.DMA,
            pltpu.SemaphoreType.DMA,
            pltpu.SemaphoreType.DMA,
            pltpu.SemaphoreType.DMA,
            pltpu.SemaphoreType.DMA,
            pltpu.SemaphoreType.DMA,
            pltpu.SemaphoreType.DMA,
            pltpu.SemaphoreType.DMA,
        ],
    )
    out1d = run(h_blk, grid_flat, source)
    return (
        out1d.reshape(2, NBATCH, 8, B)
        .transpose(1, 3, 0, 2)
        .reshape(N_REFLN, D)
    )
